# Initial kernel scaffold; baseline (speedup 1.0000x reference)
import jax
import jax.numpy as jnp
from jax.experimental import pallas as pl


def kernel(x, edge_index, edge_attr, batch_ids, c1_w1, c1_b1, c1_w2, c2_w1, c2_b1, c2_w2, l1_w, l1_b, l2_w, l2_b):
    raise NotImplementedError("write your pallas kernel here")


# speedup vs baseline: 13.7612x; 13.7612x over previous
import jax
import jax.numpy as jnp
from jax import lax
from jax.experimental import pallas as pl
from jax.experimental.pallas import tpu as pltpu
from jax.experimental.pallas import tpu_sc as plsc

N = 100000
E = 3200000
G = 64

NC = 2
NS = 16
NW = NC * NS

R = 1024
NB = 98
NP = NB * R
NP_T = NP // NS
NP_C = NP_T // 8

ROWS = 25600
EP = ROWS * 128
ROWS_PT = ROWS // NW

GA = 8
NGA = ROWS_PT // GA

_MESH = plsc.VectorSubcoreMesh(
    core_axis_name="c", subcore_axis_name="s", num_cores=NC, num_subcores=NS
)
_CP = pltpu.CompilerParams(needs_layout_passes=False, use_tc_tiling_on_sc=False)


def _init_accum(zeros16_hbm, stage, accum, s):
    pltpu.sync_copy(zeros16_hbm.at[pl.ds(0, NP_C)], stage)
    for k in range(8):
        pltpu.sync_copy(stage, accum.at[pl.ds(s * NP_T + k * NP_C, NP_C)])


def _export_accum(accum, stage, out_hbm, c, s):
    for k in range(8):
        pltpu.sync_copy(accum.at[pl.ds(s * NP_T + k * NP_C, NP_C)], stage)
        pltpu.sync_copy(
            stage, out_hbm.at[pl.ds(c * NP + s * NP_T + k * NP_C, NP_C)])


def _hist_body(row_hbm, col_hbm, zeros16_hbm, ones10_hbm, ones01_hbm,
               degcnt_hbm, rowbuf, colbuf, v10, v01, stage, accum, sem):
    c = lax.axis_index("c")
    s = lax.axis_index("s")
    w = c * NS + s
    pltpu.sync_copy(ones10_hbm, v10)
    pltpu.sync_copy(ones01_hbm, v01)
    _init_accum(zeros16_hbm, stage, accum, s)
    plsc.subcore_barrier()
    rbase = w * ROWS_PT

    def grp(g, carry):
        base = rbase + g * GA
        pltpu.sync_copy(row_hbm.at[pl.ds(base, GA)], rowbuf)
        pltpu.sync_copy(col_hbm.at[pl.ds(base, GA)], colbuf)
        descs = []
        for j in range(GA):
            descs.append(
                pltpu.async_copy(v10, accum.at[rowbuf.at[j]], sem, add=True))
            descs.append(
                pltpu.async_copy(v01, accum.at[colbuf.at[j]], sem, add=True))
        for d in descs:
            d.wait()
        return carry

    lax.fori_loop(0, NGA, grp, 0)
    plsc.subcore_barrier()
    _export_accum(accum, stage, degcnt_hbm, c, s)


_hist_kernel = pl.kernel(
    _hist_body,
    out_type=jax.ShapeDtypeStruct((2 * NP, 16), jnp.float32),
    mesh=_MESH,
    scratch_types=[
        pltpu.VMEM((GA, 128), jnp.int32),
        pltpu.VMEM((GA, 128), jnp.int32),
        pltpu.VMEM((128, 16), jnp.float32),
        pltpu.VMEM((128, 16), jnp.float32),
        pltpu.VMEM((NP_C, 16), jnp.float32),
        pltpu.VMEM_SHARED((NP, 16), jnp.float32),
        pltpu.SemaphoreType.DMA,
    ],
    compiler_params=_CP,
)


def _sgather_body(row_hbm, dis_hbm, s_hbm, rowbuf, sbuf, dis_tab):
    c = lax.axis_index("c")
    s = lax.axis_index("s")
    w = c * NS + s
    pltpu.sync_copy(dis_hbm, dis_tab)
    rbase = w * ROWS_PT

    def grp(g, carry):
        base = rbase + g * GA
        pltpu.sync_copy(row_hbm.at[pl.ds(base, GA)], rowbuf)
        for j in range(GA):
            for i in range(8):
                ridx = rowbuf[j, pl.ds(i * 16, 16)]
                sbuf[j, pl.ds(i * 16, 16)] = plsc.load_gather(dis_tab, [ridx])
        pltpu.sync_copy(sbuf, s_hbm.at[pl.ds(base, GA)])
        return carry

    lax.fori_loop(0, NGA, grp, 0)


_sgather_kernel = pl.kernel(
    _sgather_body,
    out_type=jax.ShapeDtypeStruct((ROWS, 128), jnp.float32),
    mesh=_MESH,
    scratch_types=[
        pltpu.VMEM((GA, 128), jnp.int32),
        pltpu.VMEM((GA, 128), jnp.float32),
        pltpu.VMEM((NP,), jnp.float32),
    ],
    compiler_params=_CP,
)


def _t_body(s_hbm, col_hbm, ea_hbm, zeros16_hbm, t_hbm,
            sbuf, colbuf, eabuf, tvals, accumt, sems):
    c = lax.axis_index("c")
    s = lax.axis_index("s")
    w = c * NS + s
    stage = tvals.at[pl.ds(0, NP_C)]
    _init_accum(zeros16_hbm, stage, accumt, s)
    pltpu.sync_copy(zeros16_hbm.at[pl.ds(0, GA * 128)], tvals)
    plsc.subcore_barrier()
    rbase = w * ROWS_PT
    iota = lax.iota(jnp.int32, 16)
    z16 = iota * 0
    o16 = z16 + 1

    def grp(g, carry):
        base = rbase + g * GA
        pltpu.sync_copy(s_hbm.at[pl.ds(base, GA)], sbuf)
        pltpu.sync_copy(col_hbm.at[pl.ds(base, GA)], colbuf)
        pltpu.sync_copy(ea_hbm.at[pl.ds(base, GA)], eabuf)
        for j in range(GA):
            jv = z16 + j
            for i in range(8):
                lanes = i * 16 + iota
                sv = sbuf[j, pl.ds(i * 16, 16)]
                a0 = plsc.load_gather(eabuf, [jv, 2 * lanes])
                a1 = plsc.load_gather(eabuf, [jv, 2 * lanes + 1])
                rr = j * 128 + lanes
                plsc.store_scatter(tvals, [rr, z16], sv * a0)
                plsc.store_scatter(tvals, [rr, o16], sv * a1)
        sd = []
        for j in range(GA):
            sd.append(pltpu.async_copy(
                tvals.at[pl.ds(j * 128, 128)], accumt.at[colbuf.at[j]],
                sems, add=True))
        for d in sd:
            d.wait()
        return carry

    lax.fori_loop(0, NGA, grp, 0)
    plsc.subcore_barrier()
    _export_accum(accumt, tvals.at[pl.ds(0, NP_C)], t_hbm, c, s)


_t_kernel = pl.kernel(
    _t_body,
    out_type=jax.ShapeDtypeStruct((2 * NP, 16), jnp.float32),
    mesh=_MESH,
    scratch_types=[
        pltpu.VMEM((GA, 128), jnp.float32),
        pltpu.VMEM((GA, 128), jnp.int32),
        pltpu.VMEM((GA, 256), jnp.float32),
        pltpu.VMEM((GA * 128, 16), jnp.float32),
        pltpu.VMEM_SHARED((NP, 16), jnp.float32),
        pltpu.SemaphoreType.DMA,
    ],
    compiler_params=_CP,
)


def _d_body(row_hbm, col_hbm, pt_hbm, zeros16_hbm, sum_hbm,
            rowbuf, colbuf, gath, accum, semg, sems):
    c = lax.axis_index("c")
    s = lax.axis_index("s")
    w = c * NS + s
    stage = gath.at[pl.ds(0, NP_C)]
    _init_accum(zeros16_hbm, stage, accum, s)
    plsc.subcore_barrier()
    rbase = w * ROWS_PT

    def grp(g, carry):
        base = rbase + g * GA
        pltpu.sync_copy(row_hbm.at[pl.ds(base, GA)], rowbuf)
        pltpu.sync_copy(col_hbm.at[pl.ds(base, GA)], colbuf)
        gd = []
        for j in range(GA):
            gd.append(pltpu.async_copy(
                pt_hbm.at[rowbuf.at[j]], gath.at[pl.ds(j * 128, 128)], semg))
        for d in gd:
            d.wait()
        sd = []
        for j in range(GA):
            sd.append(pltpu.async_copy(
                gath.at[pl.ds(j * 128, 128)], accum.at[colbuf.at[j]],
                sems, add=True))
        for d in sd:
            d.wait()
        return carry

    lax.fori_loop(0, NGA, grp, 0)
    plsc.subcore_barrier()
    _export_accum(accum, stage, sum_hbm, c, s)


_d_kernel = pl.kernel(
    _d_body,
    out_type=jax.ShapeDtypeStruct((2 * NP, 16), jnp.float32),
    mesh=_MESH,
    scratch_types=[
        pltpu.VMEM((GA, 128), jnp.int32),
        pltpu.VMEM((GA, 128), jnp.int32),
        pltpu.VMEM((GA * 128, 16), jnp.float32),
        pltpu.VMEM_SHARED((NP, 16), jnp.float32),
        pltpu.SemaphoreType.DMA,
        pltpu.SemaphoreType.DMA,
    ],
    compiler_params=_CP,
)


def _tc1_body(x_ref, dc_ref, w1_ref, b1_ref, w2a_ref,
              dis_ref, scale_ref, cen_ref, pt_ref):
    x = x_ref[...]
    dc = dc_ref[...]
    deg = dc[0, :, 0:1] + dc[1, :, 0:1]
    cnt = dc[0, :, 1:2] + dc[1, :, 1:2]
    dis = jnp.where(deg > 0.0, lax.rsqrt(jnp.maximum(deg, 1e-30)), 0.0)
    scale = dis / jnp.maximum(cnt, 1.0)
    cen = jnp.dot(x, w1_ref[...], preferred_element_type=jnp.float32)
    cen = cen + b1_ref[...]
    p = jnp.dot(x, w2a_ref[...], preferred_element_type=jnp.float32)
    dis_ref[...] = dis
    scale_ref[...] = scale
    cen_ref[...] = cen
    pt_ref[...] = dis * p


_tc1 = pl.pallas_call(
    _tc1_body,
    grid=(NB,),
    in_specs=[
        pl.BlockSpec((R, 3), lambda b: (b, 0)),
        pl.BlockSpec((2, R, 16), lambda b: (0, b, 0)),
        pl.BlockSpec((3, 16), lambda b: (0, 0)),
        pl.BlockSpec((1, 16), lambda b: (0, 0)),
        pl.BlockSpec((3, 16), lambda b: (0, 0)),
    ],
    out_specs=[
        pl.BlockSpec((R, 1), lambda b: (b, 0)),
        pl.BlockSpec((R, 1), lambda b: (b, 0)),
        pl.BlockSpec((R, 16), lambda b: (b, 0)),
        pl.BlockSpec((R, 16), lambda b: (b, 0)),
    ],
    out_shape=[
        jax.ShapeDtypeStruct((NP, 1), jnp.float32),
        jax.ShapeDtypeStruct((NP, 1), jnp.float32),
        jax.ShapeDtypeStruct((NP, 16), jnp.float32),
        jax.ShapeDtypeStruct((NP, 16), jnp.float32),
    ],
)


def _tc2_body(cen_ref, dis_ref, scale_ref, sum_ref, t_ref, wb_ref,
              w1_ref, b1_ref, w2a_ref, cen2_ref, pt2_ref):
    rowsum = sum_ref[0] + sum_ref[1]
    t = t_ref[...]
    t0 = t[0, :, 0:1] + t[1, :, 0:1]
    t1 = t[0, :, 1:2] + t[1, :, 1:2]
    wb = wb_ref[...]
    aggr = scale_ref[...] * (rowsum + t0 * wb[0:1, :] + t1 * wb[1:2, :])
    h1 = jnp.maximum(cen_ref[...] + aggr, 0.0)
    cen2 = jnp.dot(h1, w1_ref[...], preferred_element_type=jnp.float32)
    cen2_ref[...] = cen2 + b1_ref[...]
    pt2_ref[...] = dis_ref[...] * jnp.dot(
        h1, w2a_ref[...], preferred_element_type=jnp.float32)


_tc2 = pl.pallas_call(
    _tc2_body,
    grid=(NB,),
    in_specs=[
        pl.BlockSpec((R, 16), lambda b: (b, 0)),
        pl.BlockSpec((R, 1), lambda b: (b, 0)),
        pl.BlockSpec((R, 1), lambda b: (b, 0)),
        pl.BlockSpec((2, R, 16), lambda b: (0, b, 0)),
        pl.BlockSpec((2, R, 16), lambda b: (0, b, 0)),
        pl.BlockSpec((2, 16), lambda b: (0, 0)),
        pl.BlockSpec((16, 16), lambda b: (0, 0)),
        pl.BlockSpec((1, 16), lambda b: (0, 0)),
        pl.BlockSpec((16, 16), lambda b: (0, 0)),
    ],
    out_specs=[
        pl.BlockSpec((R, 16), lambda b: (b, 0)),
        pl.BlockSpec((R, 16), lambda b: (b, 0)),
    ],
    out_shape=[
        jax.ShapeDtypeStruct((NP, 16), jnp.float32),
        jax.ShapeDtypeStruct((NP, 16), jnp.float32),
    ],
)


def _tc3_body(cen2_ref, scale_ref, sum_ref, t_ref, wb_ref, bid_ref,
              l1w_ref, l1b_ref, l2w_ref, l2b_ref, out_ref, acc_ref, cnt_ref):
    b = pl.program_id(0)

    @pl.when(b == 0)
    def _():
        acc_ref[...] = jnp.zeros_like(acc_ref)
        cnt_ref[...] = jnp.zeros_like(cnt_ref)

    rowsum = sum_ref[0] + sum_ref[1]
    t = t_ref[...]
    t0 = t[0, :, 0:1] + t[1, :, 0:1]
    t1 = t[0, :, 1:2] + t[1, :, 1:2]
    wb = wb_ref[...]
    aggr = scale_ref[...] * (rowsum + t0 * wb[0:1, :] + t1 * wb[1:2, :])
    h2 = jnp.maximum(cen2_ref[...] + aggr, 0.0)
    ids = bid_ref[...]
    gidx = lax.broadcasted_iota(jnp.int32, (R, G), 1)
    oh = (ids == gidx).astype(jnp.float32)
    acc_ref[...] += lax.dot_general(
        oh, h2, (((0,), (0,)), ((), ())), preferred_element_type=jnp.float32)
    cnt_ref[...] += lax.dot_general(
        oh, jnp.ones((R, 1), jnp.float32), (((0,), (0,)), ((), ())),
        preferred_element_type=jnp.float32)

    @pl.when(b == NB - 1)
    def _():
        gm = acc_ref[...] / jnp.maximum(cnt_ref[...], 1.0)
        z = jnp.maximum(
            jnp.dot(gm, l1w_ref[...], preferred_element_type=jnp.float32)
            + l1b_ref[...], 0.0)
        out_ref[...] = jnp.dot(
            z, l2w_ref[...], preferred_element_type=jnp.float32) + l2b_ref[...]


_tc3 = pl.pallas_call(
    _tc3_body,
    grid=(NB,),
    in_specs=[
        pl.BlockSpec((R, 16), lambda b: (b, 0)),
        pl.BlockSpec((R, 1), lambda b: (b, 0)),
        pl.BlockSpec((2, R, 16), lambda b: (0, b, 0)),
        pl.BlockSpec((2, R, 16), lambda b: (0, b, 0)),
        pl.BlockSpec((2, 16), lambda b: (0, 0)),
        pl.BlockSpec((R, 1), lambda b: (b, 0)),
        pl.BlockSpec((16, 16), lambda b: (0, 0)),
        pl.BlockSpec((1, 16), lambda b: (0, 0)),
        pl.BlockSpec((16, 2), lambda b: (0, 0)),
        pl.BlockSpec((1, 2), lambda b: (0, 0)),
    ],
    out_specs=pl.BlockSpec((G, 2), lambda b: (0, 0)),
    out_shape=jax.ShapeDtypeStruct((G, 2), jnp.float32),
    scratch_shapes=[
        pltpu.VMEM((G, 16), jnp.float32),
        pltpu.VMEM((G, 1), jnp.float32),
    ],
)


def kernel(x, edge_index, edge_attr, batch_ids, c1_w1, c1_b1, c1_w2,
           c2_w1, c2_b1, c2_w2, l1_w, l1_b, l2_w, l2_b):
    f32 = jnp.float32
    xp = jnp.zeros((NP, 3), f32).at[:N].set(x)
    row = jnp.concatenate(
        [edge_index[0], jnp.full((EP - E,), N, jnp.int32)]).reshape(ROWS, 128)
    col = jnp.concatenate(
        [edge_index[1], jnp.full((EP - E,), N, jnp.int32)]).reshape(ROWS, 128)
    eap = jnp.concatenate(
        [edge_attr, jnp.zeros((EP - E, 2), f32)]).reshape(ROWS, 256)
    bid = jnp.concatenate(
        [batch_ids, jnp.full((NP - N,), G, jnp.int32)]).reshape(NP, 1)
    zeros16 = jnp.zeros((NP, 16), f32)
    ones10 = jnp.zeros((128, 16), f32).at[:, 0].set(1.0)
    ones01 = jnp.zeros((128, 16), f32).at[:, 1].set(1.0)

    degcnt = _hist_kernel(row, col, zeros16, ones10, ones01).reshape(2, NP, 16)

    dis, scale, cen1, pt1 = _tc1(xp, degcnt, c1_w1, c1_b1.reshape(1, 16),
                                 c1_w2[:3])

    s2d = _sgather_kernel(row, dis.reshape(NP))
    tp = _t_kernel(s2d, col, eap, zeros16).reshape(2, NP, 16)
    sum1 = _d_kernel(row, col, pt1, zeros16).reshape(2, NP, 16)

    cen2, pt2 = _tc2(cen1, dis, scale, sum1, tp, c1_w2[3:5], c2_w1,
                     c2_b1.reshape(1, 16), c2_w2[:16])

    sum2 = _d_kernel(row, col, pt2, zeros16).reshape(2, NP, 16)

    out = _tc3(cen2, scale, sum2, tp, c2_w2[16:18], bid, l1_w,
               l1_b.reshape(1, 16), l2_w, l2_b.reshape(1, 2))
    return out
